# initial kernel scaffold (unmeasured)
import jax
import jax.numpy as jnp
from jax import lax
from jax.experimental import pallas as pl
from jax.experimental.pallas import tpu as pltpu

N_DEV = 8
M_PER = 512
N_PER = 1024
K = 4096


def kernel(x, w_mat, scale_x, scale_w):
    x8 = x.astype(jnp.float8_e5m2)
    w8 = w_mat.astype(jnp.float8_e5m2)
    scale = (scale_x[0] * scale_w[0]).reshape(1, 1).astype(jnp.float32)

    def body(x_ref, w_ref, s_ref, out_ref, sendbuf, send_sems, recv_sems):
        my = lax.axis_index("i")
        s = s_ref[0, 0]

        def block(col):
            acc = jnp.dot(
                x_ref[...],
                w_ref[:, pl.ds(col * N_PER, N_PER)],
                preferred_element_type=jnp.float32,
            )
            y = acc * s
            return y * jax.nn.sigmoid(y)

        sends = []
        for t in range(1, N_DEV):
            j = (my + t) % N_DEV
            z = block(j)
            slot = (t - 1) % 2
            if t >= 3:
                sends[t - 3].wait_send()
            sendbuf[slot, :, :] = z
            rdma = pltpu.make_async_remote_copy(
                src_ref=sendbuf.at[slot],
                dst_ref=out_ref.at[pl.ds(my * M_PER, M_PER), :],
                send_sem=send_sems.at[slot],
                recv_sem=recv_sems.at[t - 1],
                device_id=(j,),
                device_id_type=pl.DeviceIdType.MESH,
            )
            rdma.start()
            sends.append(rdma)

        out_ref[pl.ds(my * M_PER, M_PER), :] = block(my)

        sends[-2].wait_send()
        sends[-1].wait_send()

        for t in range(1, N_DEV):
            src = (my - t) % N_DEV
            recv = pltpu.make_async_remote_copy(
                src_ref=sendbuf.at[0],
                dst_ref=out_ref.at[pl.ds(src * M_PER, M_PER), :],
                send_sem=send_sems.at[0],
                recv_sem=recv_sems.at[t - 1],
                device_id=(src,),
                device_id_type=pl.DeviceIdType.MESH,
            )
            recv.wait_recv()

    return pl.pallas_call(
        body,
        out_shape=jax.ShapeDtypeStruct((N_DEV * M_PER, N_PER), jnp.float32),
        in_specs=[
            pl.BlockSpec(memory_space=pltpu.VMEM),
            pl.BlockSpec(memory_space=pltpu.VMEM),
            pl.BlockSpec(memory_space=pltpu.SMEM),
        ],
        out_specs=pl.BlockSpec(memory_space=pltpu.VMEM),
        scratch_shapes=[
            pltpu.VMEM((2, M_PER, N_PER), jnp.float32),
            pltpu.SemaphoreType.DMA((2,)),
            pltpu.SemaphoreType.DMA((7,)),
        ],
        compiler_params=pltpu.CompilerParams(collective_id=0),
    )(x8, w8, scale)


# baseline (device time: 247134 ns/iter reference)
import jax
import jax.numpy as jnp
from jax import lax
from jax.experimental import pallas as pl
from jax.experimental.pallas import tpu as pltpu

N_DEV = 8
M_PER = 512
N_PER = 1024
K = 4096


def kernel(x, w_mat, scale_x, scale_w):
    x8 = x.astype(jnp.float8_e5m2)
    w8 = w_mat.astype(jnp.float8_e5m2)
    scale = (scale_x[0] * scale_w[0]).reshape(1, 1).astype(jnp.float32)

    def body(x_ref, w_ref, s_ref, out_ref, sendbuf, send_sems, recv_sems):
        my = lax.axis_index("i")
        s = s_ref[0, 0]

        def block(col):
            acc = jnp.dot(
                x_ref[...],
                w_ref[:, pl.ds(col * N_PER, N_PER)],
                preferred_element_type=jnp.float32,
            )
            y = acc * s
            return y * jax.nn.sigmoid(y)

        sends = []
        for t in range(1, N_DEV):
            j = (my + t) % N_DEV
            z = block(j)
            slot = (t - 1) % 2
            if t >= 3:
                sends[t - 3].wait_send()
            sendbuf[slot, :, :] = z
            rdma = pltpu.make_async_remote_copy(
                src_ref=sendbuf.at[slot],
                dst_ref=out_ref.at[pl.ds(my * M_PER, M_PER), :],
                send_sem=send_sems.at[slot],
                recv_sem=recv_sems.at[t - 1],
                device_id=(j,),
                device_id_type=pl.DeviceIdType.MESH,
            )
            rdma.start()
            sends.append(rdma)

        out_ref[pl.ds(my * M_PER, M_PER), :] = block(my)

        sends[-2].wait_send()
        sends[-1].wait_send()

        for t in range(1, N_DEV):
            src = (my - t) % N_DEV
            recv = pltpu.make_async_remote_copy(
                src_ref=sendbuf.at[0],
                dst_ref=out_ref.at[pl.ds(src * M_PER, M_PER), :],
                send_sem=send_sems.at[0],
                recv_sem=recv_sems.at[t - 1],
                device_id=(src,),
                device_id_type=pl.DeviceIdType.MESH,
            )
            recv.wait_recv()

    return pl.pallas_call(
        body,
        out_shape=jax.ShapeDtypeStruct((N_DEV * M_PER, N_PER), jnp.float32),
        in_specs=[
            pl.BlockSpec(memory_space=pltpu.VMEM),
            pl.BlockSpec(memory_space=pltpu.VMEM),
            pl.BlockSpec(memory_space=pltpu.SMEM),
        ],
        out_specs=pl.BlockSpec(memory_space=pltpu.VMEM),
        scratch_shapes=[
            pltpu.VMEM((2, M_PER, N_PER), jnp.float32),
            pltpu.SemaphoreType.DMA((2,)),
            pltpu.SemaphoreType.DMA((7,)),
        ],
        compiler_params=pltpu.CompilerParams(
            vmem_limit_bytes=100 * 1024 * 1024,
        ),
    )(x8, w8, scale)


# device time: 93674 ns/iter; 2.6382x vs baseline; 2.6382x over previous
import os

import jax
import jax.numpy as jnp
from jax import lax
from jax.experimental import pallas as pl
from jax.experimental.pallas import tpu as pltpu

N_DEV = 8
M_PER = 512
N_PER = 1024
K = 4096

_KVAR = os.environ.get("KVAR", "full")
_DO_COMPUTE = _KVAR != "comm"
_DO_COMM = _KVAR != "compute"


def kernel(x, w_mat, scale_x, scale_w):
    x8 = x.astype(jnp.float8_e5m2)
    w8 = w_mat.astype(jnp.float8_e5m2)
    scale = (scale_x[0] * scale_w[0]).reshape(1, 1).astype(jnp.float32)

    def body(x_ref, w_ref, s_ref, out_ref, sendbuf, send_sems, recv_sems):
        my = lax.axis_index("i")
        s = s_ref[0, 0]

        def block(col):
            acc = jnp.dot(
                x_ref[...],
                w_ref[:, pl.ds(col * N_PER, N_PER)],
                preferred_element_type=jnp.float32,
            )
            y = acc * s
            return y * jax.nn.sigmoid(y)

        sends = []
        for t in range(1, N_DEV):
            j = (my + t) % N_DEV
            slot = t - 1
            if _DO_COMPUTE:
                sendbuf[slot, :, :] = block(j)
            else:
                sendbuf[slot, :, :] = jnp.full(
                    (M_PER, N_PER), 0.5, dtype=jnp.float32
                )
            if _DO_COMM:
                rdma = pltpu.make_async_remote_copy(
                    src_ref=sendbuf.at[slot],
                    dst_ref=out_ref.at[pl.ds(my * M_PER, M_PER), :],
                    send_sem=send_sems.at[slot],
                    recv_sem=recv_sems.at[t - 1],
                    device_id=(j,),
                    device_id_type=pl.DeviceIdType.MESH,
                )
                rdma.start()
                sends.append(rdma)

        if _DO_COMPUTE:
            out_ref[pl.ds(my * M_PER, M_PER), :] = block(my)
        else:
            out_ref[pl.ds(my * M_PER, M_PER), :] = sendbuf[0, :, :]

        for rdma in sends:
            rdma.wait_send()
        if _DO_COMM:
            for t in range(1, N_DEV):
                src = (my - t) % N_DEV
                recv = pltpu.make_async_remote_copy(
                    src_ref=sendbuf.at[0],
                    dst_ref=out_ref.at[pl.ds(src * M_PER, M_PER), :],
                    send_sem=send_sems.at[0],
                    recv_sem=recv_sems.at[t - 1],
                    device_id=(src,),
                    device_id_type=pl.DeviceIdType.MESH,
                )
                recv.wait_recv()

    return pl.pallas_call(
        body,
        out_shape=jax.ShapeDtypeStruct((N_DEV * M_PER, N_PER), jnp.float32),
        in_specs=[
            pl.BlockSpec(memory_space=pltpu.VMEM),
            pl.BlockSpec(memory_space=pltpu.VMEM),
            pl.BlockSpec(memory_space=pltpu.SMEM),
        ],
        out_specs=pl.BlockSpec(memory_space=pltpu.VMEM),
        scratch_shapes=[
            pltpu.VMEM((N_DEV - 1, M_PER, N_PER), jnp.float32),
            pltpu.SemaphoreType.DMA((N_DEV - 1,)),
            pltpu.SemaphoreType.DMA((N_DEV - 1,)),
        ],
        compiler_params=pltpu.CompilerParams(
            vmem_limit_bytes=128 * 1024 * 1024,
        ),
    )(x8, w8, scale)
